# baseline (speedup 1.0000x reference)
import math

import jax
import jax.numpy as jnp
from jax.experimental import pallas as pl

_D = 1024
_H = 16
_DK = _D // _H
_S = 2048
_K = max(1, int(_S * 0.3))
_SCALE = 1.0 / math.sqrt(_DK)
_BQ = 256
_INT_MIN = -(2 ** 31)


def _qkv_body(x_ref, wq_ref, wk_ref, wv_ref, bq_ref, bk_ref, bv_ref,
              q_ref, k_ref, v_ref):
    x = x_ref[...]
    q_ref[...] = jnp.dot(x, wq_ref[...], preferred_element_type=jnp.float32) + bq_ref[...]
    k_ref[...] = jnp.dot(x, wk_ref[...], preferred_element_type=jnp.float32) + bk_ref[...]
    v_ref[...] = jnp.dot(x, wv_ref[...], preferred_element_type=jnp.float32) + bv_ref[...]


def _attn_body(q_ref, k_ref, v_ref, attn_ref, ctx_ref):
    q = q_ref[0]
    k = k_ref[0]
    s = jax.lax.dot_general(
        q, k, (((1,), (1,)), ((), ())),
        preferred_element_type=jnp.float32) * _SCALE

    bits = jax.lax.bitcast_convert_type(s, jnp.int32)
    ikey = jnp.where(bits < 0, bits ^ jnp.int32(0x7FFFFFFF), bits)

    kb = jax.lax.bitcast_convert_type(
        bits & jnp.int32(-0x10000), jnp.float32).astype(jnp.bfloat16)
    kf = jnp.float32(_K)

    def _count16(code):
        ihi = jnp.where(code >= 0, code, code ^ jnp.int32(0x7FFF))
        tf = jax.lax.bitcast_convert_type(ihi << 16, jnp.float32)
        pb = (kb >= tf.astype(jnp.bfloat16)).astype(jnp.bfloat16)
        part = jnp.sum(pb.reshape(_BQ, _S // 128, 128), axis=1)
        return jnp.sum(part.astype(jnp.float32), axis=1, keepdims=True)

    tc = jnp.where(_count16(jnp.zeros((_BQ, 1), jnp.int32)) >= kf,
                   0, -0x8000).astype(jnp.int32)
    for b in range(14, -1, -1):
        trial = tc | jnp.int32(1 << b)
        tc = jnp.where(_count16(trial) >= kf, trial, tc)

    thr = tc << 16
    for b in range(15, 7, -1):
        trial = thr | jnp.int32(1 << b)
        cnt = jnp.sum((ikey >= trial).astype(jnp.int32), axis=1, keepdims=True)
        thr = jnp.where(cnt >= _K, trial, thr)

    mask = ikey >= thr
    m = jnp.max(s, axis=1, keepdims=True)
    p = jnp.where(mask, jnp.exp(s - m), 0.0)
    denom = jnp.sum(p, axis=1, keepdims=True)
    a = p / denom
    attn_ref[0] = a
    ctx_ref[0] = jnp.dot(a, v_ref[0], preferred_element_type=jnp.float32)


def _proj_body(c_ref, w_ref, b_ref, o_ref):
    o_ref[...] = jnp.dot(c_ref[...], w_ref[...],
                         preferred_element_type=jnp.float32) + b_ref[...]


def kernel(x, W_q, b_q, W_k, b_k, W_v, b_v, W_o, b_o):
    x2 = x.reshape(_S, _D)
    wqt, wkt, wvt, wot = W_q.T, W_k.T, W_v.T, W_o.T
    bq2 = b_q.reshape(1, _D)
    bk2 = b_k.reshape(1, _D)
    bv2 = b_v.reshape(1, _D)
    bo2 = b_o.reshape(1, _D)
    nb = _S // _BQ

    q, kk, v = pl.pallas_call(
        _qkv_body,
        grid=(nb,),
        in_specs=[
            pl.BlockSpec((_BQ, _D), lambda i: (i, 0)),
            pl.BlockSpec((_D, _D), lambda i: (0, 0)),
            pl.BlockSpec((_D, _D), lambda i: (0, 0)),
            pl.BlockSpec((_D, _D), lambda i: (0, 0)),
            pl.BlockSpec((1, _D), lambda i: (0, 0)),
            pl.BlockSpec((1, _D), lambda i: (0, 0)),
            pl.BlockSpec((1, _D), lambda i: (0, 0)),
        ],
        out_specs=[
            pl.BlockSpec((_BQ, _D), lambda i: (i, 0)),
            pl.BlockSpec((_BQ, _D), lambda i: (i, 0)),
            pl.BlockSpec((_BQ, _D), lambda i: (i, 0)),
        ],
        out_shape=[jax.ShapeDtypeStruct((_S, _D), jnp.float32)] * 3,
    )(x2, wqt, wkt, wvt, bq2, bk2, bv2)

    q3 = q.reshape(_S, _H, _DK).transpose(1, 0, 2)
    k3 = kk.reshape(_S, _H, _DK).transpose(1, 0, 2)
    v3 = v.reshape(_S, _H, _DK).transpose(1, 0, 2)

    attn, ctx = pl.pallas_call(
        _attn_body,
        grid=(_H, nb),
        in_specs=[
            pl.BlockSpec((1, _BQ, _DK), lambda h, i: (h, i, 0)),
            pl.BlockSpec((1, _S, _DK), lambda h, i: (h, 0, 0)),
            pl.BlockSpec((1, _S, _DK), lambda h, i: (h, 0, 0)),
        ],
        out_specs=[
            pl.BlockSpec((1, _BQ, _S), lambda h, i: (h, i, 0)),
            pl.BlockSpec((1, _BQ, _DK), lambda h, i: (h, i, 0)),
        ],
        out_shape=[
            jax.ShapeDtypeStruct((_H, _S, _S), jnp.float32),
            jax.ShapeDtypeStruct((_H, _S, _DK), jnp.float32),
        ],
    )(q3, k3, v3)

    ctx2 = ctx.transpose(1, 0, 2).reshape(_S, _D)

    out = pl.pallas_call(
        _proj_body,
        grid=(nb,),
        in_specs=[
            pl.BlockSpec((_BQ, _D), lambda i: (i, 0)),
            pl.BlockSpec((_D, _D), lambda i: (0, 0)),
            pl.BlockSpec((1, _D), lambda i: (0, 0)),
        ],
        out_specs=pl.BlockSpec((_BQ, _D), lambda i: (i, 0)),
        out_shape=jax.ShapeDtypeStruct((_S, _D), jnp.float32),
    )(ctx2, wot, bo2)

    return (out.reshape(1, _S, _D), attn.reshape(1, _H, _S, _S))


# speedup vs baseline: 1.5790x; 1.5790x over previous
import math

import jax
import jax.numpy as jnp
from jax.experimental import pallas as pl

_D = 1024
_H = 16
_DK = _D // _H
_S = 2048
_K = max(1, int(_S * 0.3))
_SCALE = 1.0 / math.sqrt(_DK)
_BQ = 256
_INT_MIN = -(2 ** 31)


def _qkv_body(x_ref, wq_ref, wk_ref, wv_ref, bq_ref, bk_ref, bv_ref,
              q_ref, k_ref, v_ref):
    x = x_ref[...]
    q_ref[...] = jnp.dot(x, wq_ref[...], preferred_element_type=jnp.float32) + bq_ref[...]
    k_ref[...] = jnp.dot(x, wk_ref[...], preferred_element_type=jnp.float32) + bk_ref[...]
    v_ref[...] = jnp.dot(x, wv_ref[...], preferred_element_type=jnp.float32) + bv_ref[...]


def _attn_body(q_ref, k_ref, v_ref, attn_ref, ctx_ref):
    q = q_ref[0]
    k = k_ref[0]
    s = jax.lax.dot_general(
        q, k, (((1,), (1,)), ((), ())),
        preferred_element_type=jnp.float32) * _SCALE

    bits = jax.lax.bitcast_convert_type(s, jnp.int32)
    ikey = jnp.where(bits < 0, bits ^ jnp.int32(0x7FFFFFFF), bits)

    kb = jax.lax.bitcast_convert_type(
        bits & jnp.int32(-0x10000), jnp.float32).astype(jnp.bfloat16)
    kf = jnp.float32(_K)

    def _count16(code):
        ihi = jnp.where(code >= 0, code, code ^ jnp.int32(0x7FFF))
        tf = jax.lax.bitcast_convert_type(ihi << 16, jnp.float32)
        pb = (kb >= tf.astype(jnp.bfloat16)).astype(jnp.bfloat16)
        w = _S // 2
        while w >= 128:
            pb = pb[:, :w] + pb[:, w:]
            w //= 2
        return jnp.sum(pb.astype(jnp.float32), axis=1, keepdims=True)

    tc = jnp.where(_count16(jnp.zeros((_BQ, 1), jnp.int32)) >= kf,
                   0, -0x8000).astype(jnp.int32)
    for b in range(14, -1, -1):
        trial = tc | jnp.int32(1 << b)
        tc = jnp.where(_count16(trial) >= kf, trial, tc)

    thr = tc << 16
    for b in range(15, 7, -1):
        trial = thr | jnp.int32(1 << b)
        cnt = jnp.sum((ikey >= trial).astype(jnp.int32), axis=1, keepdims=True)
        thr = jnp.where(cnt >= _K, trial, thr)

    mask = ikey >= thr
    m = jnp.max(s, axis=1, keepdims=True)
    p = jnp.where(mask, jnp.exp(s - m), 0.0)
    denom = jnp.sum(p, axis=1, keepdims=True)
    a = p / denom
    attn_ref[0] = a
    ctx_ref[0] = jnp.dot(a, v_ref[0], preferred_element_type=jnp.float32)


def _proj_body(c_ref, w_ref, b_ref, o_ref):
    o_ref[...] = jnp.dot(c_ref[...], w_ref[...],
                         preferred_element_type=jnp.float32) + b_ref[...]


def kernel(x, W_q, b_q, W_k, b_k, W_v, b_v, W_o, b_o):
    x2 = x.reshape(_S, _D)
    wqt, wkt, wvt, wot = W_q.T, W_k.T, W_v.T, W_o.T
    bq2 = b_q.reshape(1, _D)
    bk2 = b_k.reshape(1, _D)
    bv2 = b_v.reshape(1, _D)
    bo2 = b_o.reshape(1, _D)
    nb = _S // _BQ

    q, kk, v = pl.pallas_call(
        _qkv_body,
        grid=(nb,),
        in_specs=[
            pl.BlockSpec((_BQ, _D), lambda i: (i, 0)),
            pl.BlockSpec((_D, _D), lambda i: (0, 0)),
            pl.BlockSpec((_D, _D), lambda i: (0, 0)),
            pl.BlockSpec((_D, _D), lambda i: (0, 0)),
            pl.BlockSpec((1, _D), lambda i: (0, 0)),
            pl.BlockSpec((1, _D), lambda i: (0, 0)),
            pl.BlockSpec((1, _D), lambda i: (0, 0)),
        ],
        out_specs=[
            pl.BlockSpec((_BQ, _D), lambda i: (i, 0)),
            pl.BlockSpec((_BQ, _D), lambda i: (i, 0)),
            pl.BlockSpec((_BQ, _D), lambda i: (i, 0)),
        ],
        out_shape=[jax.ShapeDtypeStruct((_S, _D), jnp.float32)] * 3,
    )(x2, wqt, wkt, wvt, bq2, bk2, bv2)

    q3 = q.reshape(_S, _H, _DK).transpose(1, 0, 2)
    k3 = kk.reshape(_S, _H, _DK).transpose(1, 0, 2)
    v3 = v.reshape(_S, _H, _DK).transpose(1, 0, 2)

    attn, ctx = pl.pallas_call(
        _attn_body,
        grid=(_H, nb),
        in_specs=[
            pl.BlockSpec((1, _BQ, _DK), lambda h, i: (h, i, 0)),
            pl.BlockSpec((1, _S, _DK), lambda h, i: (h, 0, 0)),
            pl.BlockSpec((1, _S, _DK), lambda h, i: (h, 0, 0)),
        ],
        out_specs=[
            pl.BlockSpec((1, _BQ, _S), lambda h, i: (h, i, 0)),
            pl.BlockSpec((1, _BQ, _DK), lambda h, i: (h, i, 0)),
        ],
        out_shape=[
            jax.ShapeDtypeStruct((_H, _S, _S), jnp.float32),
            jax.ShapeDtypeStruct((_H, _S, _DK), jnp.float32),
        ],
    )(q3, k3, v3)

    ctx2 = ctx.transpose(1, 0, 2).reshape(_S, _D)

    out = pl.pallas_call(
        _proj_body,
        grid=(nb,),
        in_specs=[
            pl.BlockSpec((_BQ, _D), lambda i: (i, 0)),
            pl.BlockSpec((_D, _D), lambda i: (0, 0)),
            pl.BlockSpec((1, _D), lambda i: (0, 0)),
        ],
        out_specs=pl.BlockSpec((_BQ, _D), lambda i: (i, 0)),
        out_shape=jax.ShapeDtypeStruct((_S, _D), jnp.float32),
    )(ctx2, wot, bo2)

    return (out.reshape(1, _S, _D), attn.reshape(1, _H, _S, _S))


# speedup vs baseline: 2.6097x; 1.6528x over previous
import math

import jax
import jax.numpy as jnp
from jax.experimental import pallas as pl

_D = 1024
_H = 16
_DK = _D // _H
_S = 2048
_K = max(1, int(_S * 0.3))
_SCALE = 1.0 / math.sqrt(_DK)
_BQ = 512
_INT_MIN = -(2 ** 31)


def _qkv_body(x_ref, wq_ref, wk_ref, wv_ref, bq_ref, bk_ref, bv_ref,
              q_ref, k_ref, v_ref):
    x = x_ref[...]
    q_ref[...] = jnp.dot(x, wq_ref[...], preferred_element_type=jnp.float32) + bq_ref[...]
    k_ref[...] = jnp.dot(x, wk_ref[...], preferred_element_type=jnp.float32) + bk_ref[...]
    v_ref[...] = jnp.dot(x, wv_ref[...], preferred_element_type=jnp.float32) + bv_ref[...]


def _attn_body(q_ref, k_ref, v_ref, attn_ref, ctx_ref):
    q = q_ref[0]
    k = k_ref[0]
    s = jax.lax.dot_general(
        q, k, (((1,), (1,)), ((), ())),
        preferred_element_type=jnp.float32) * _SCALE

    bits = jax.lax.bitcast_convert_type(s, jnp.int32)
    ikey = jnp.where(bits < 0, bits ^ jnp.int32(0x7FFFFFFF), bits)

    cnt = jnp.sum((ikey >= 0).astype(jnp.int32), axis=1, keepdims=True)
    thr = jnp.where(cnt >= _K, 0, _INT_MIN).astype(jnp.int32)
    for b in range(30, 7, -1):
        trial = thr | jnp.int32(1 << b)
        cnt = jnp.sum((ikey >= trial).astype(jnp.int32), axis=1, keepdims=True)
        thr = jnp.where(cnt >= _K, trial, thr)

    mask = ikey >= thr
    m = jnp.max(s, axis=1, keepdims=True)
    p = jnp.where(mask, jnp.exp(s - m), 0.0)
    denom = jnp.sum(p, axis=1, keepdims=True)
    a = p / denom
    attn_ref[0] = a
    ctx_ref[0] = jnp.dot(a, v_ref[0], preferred_element_type=jnp.float32)


def _proj_body(c_ref, w_ref, b_ref, o_ref):
    o_ref[...] = jnp.dot(c_ref[...], w_ref[...],
                         preferred_element_type=jnp.float32) + b_ref[...]


def kernel(x, W_q, b_q, W_k, b_k, W_v, b_v, W_o, b_o):
    x2 = x.reshape(_S, _D)
    wqt, wkt, wvt, wot = W_q.T, W_k.T, W_v.T, W_o.T
    bq2 = b_q.reshape(1, _D)
    bk2 = b_k.reshape(1, _D)
    bv2 = b_v.reshape(1, _D)
    bo2 = b_o.reshape(1, _D)
    nb = _S // _BQ

    q, kk, v = pl.pallas_call(
        _qkv_body,
        grid=(nb,),
        in_specs=[
            pl.BlockSpec((_BQ, _D), lambda i: (i, 0)),
            pl.BlockSpec((_D, _D), lambda i: (0, 0)),
            pl.BlockSpec((_D, _D), lambda i: (0, 0)),
            pl.BlockSpec((_D, _D), lambda i: (0, 0)),
            pl.BlockSpec((1, _D), lambda i: (0, 0)),
            pl.BlockSpec((1, _D), lambda i: (0, 0)),
            pl.BlockSpec((1, _D), lambda i: (0, 0)),
        ],
        out_specs=[
            pl.BlockSpec((_BQ, _D), lambda i: (i, 0)),
            pl.BlockSpec((_BQ, _D), lambda i: (i, 0)),
            pl.BlockSpec((_BQ, _D), lambda i: (i, 0)),
        ],
        out_shape=[jax.ShapeDtypeStruct((_S, _D), jnp.float32)] * 3,
    )(x2, wqt, wkt, wvt, bq2, bk2, bv2)

    q3 = q.reshape(_S, _H, _DK).transpose(1, 0, 2)
    k3 = kk.reshape(_S, _H, _DK).transpose(1, 0, 2)
    v3 = v.reshape(_S, _H, _DK).transpose(1, 0, 2)

    attn, ctx = pl.pallas_call(
        _attn_body,
        grid=(_H, nb),
        in_specs=[
            pl.BlockSpec((1, _BQ, _DK), lambda h, i: (h, i, 0)),
            pl.BlockSpec((1, _S, _DK), lambda h, i: (h, 0, 0)),
            pl.BlockSpec((1, _S, _DK), lambda h, i: (h, 0, 0)),
        ],
        out_specs=[
            pl.BlockSpec((1, _BQ, _S), lambda h, i: (h, i, 0)),
            pl.BlockSpec((1, _BQ, _DK), lambda h, i: (h, i, 0)),
        ],
        out_shape=[
            jax.ShapeDtypeStruct((_H, _S, _S), jnp.float32),
            jax.ShapeDtypeStruct((_H, _S, _DK), jnp.float32),
        ],
    )(q3, k3, v3)

    ctx2 = ctx.transpose(1, 0, 2).reshape(_S, _D)

    out = pl.pallas_call(
        _proj_body,
        grid=(nb,),
        in_specs=[
            pl.BlockSpec((_BQ, _D), lambda i: (i, 0)),
            pl.BlockSpec((_D, _D), lambda i: (0, 0)),
            pl.BlockSpec((1, _D), lambda i: (0, 0)),
        ],
        out_specs=pl.BlockSpec((_BQ, _D), lambda i: (i, 0)),
        out_shape=jax.ShapeDtypeStruct((_S, _D), jnp.float32),
    )(ctx2, wot, bo2)

    return (out.reshape(1, _S, _D), attn.reshape(1, _H, _S, _S))


# speedup vs baseline: 3.0758x; 1.1786x over previous
import math

import jax
import jax.numpy as jnp
from jax.experimental import pallas as pl

_D = 1024
_H = 16
_DK = _D // _H
_S = 2048
_K = max(1, int(_S * 0.3))
_SCALE = 1.0 / math.sqrt(_DK)
_BQ = 512
_INT_MIN = -(2 ** 31)


def _qkv_body(x_ref, wq_ref, wk_ref, wv_ref, bq_ref, bk_ref, bv_ref,
              q_ref, k_ref, v_ref):
    x = x_ref[...]
    q_ref[...] = jnp.dot(x, wq_ref[...], preferred_element_type=jnp.float32) + bq_ref[...]
    k_ref[...] = jnp.dot(x, wk_ref[...], preferred_element_type=jnp.float32) + bk_ref[...]
    v_ref[...] = jnp.dot(x, wv_ref[...], preferred_element_type=jnp.float32) + bv_ref[...]


def _attn_body(q_ref, k_ref, v_ref, attn_ref, ctx_ref):
    q = q_ref[0]
    k = k_ref[0]
    s = jax.lax.dot_general(
        q, k, (((1,), (1,)), ((), ())),
        preferred_element_type=jnp.float32) * _SCALE

    m = jnp.max(s, axis=1, keepdims=True)
    lo = jnp.min(s, axis=1, keepdims=True)
    hi = m
    kf = jnp.float32(_K)
    for _ in range(20):
        mid = 0.5 * (lo + hi)
        cnt = jnp.sum((s >= mid).astype(jnp.float32), axis=1, keepdims=True)
        keep = cnt >= kf
        lo = jnp.where(keep, mid, lo)
        hi = jnp.where(keep, hi, mid)

    mask = s >= lo
    p = jnp.where(mask, jnp.exp(s - m), 0.0)
    denom = jnp.sum(p, axis=1, keepdims=True)
    a = p / denom
    attn_ref[0] = a
    ctx_ref[0] = jnp.dot(a, v_ref[0], preferred_element_type=jnp.float32)


def _proj_body(c_ref, w_ref, b_ref, o_ref):
    o_ref[...] = jnp.dot(c_ref[...], w_ref[...],
                         preferred_element_type=jnp.float32) + b_ref[...]


def kernel(x, W_q, b_q, W_k, b_k, W_v, b_v, W_o, b_o):
    x2 = x.reshape(_S, _D)
    wqt, wkt, wvt, wot = W_q.T, W_k.T, W_v.T, W_o.T
    bq2 = b_q.reshape(1, _D)
    bk2 = b_k.reshape(1, _D)
    bv2 = b_v.reshape(1, _D)
    bo2 = b_o.reshape(1, _D)
    nb = _S // _BQ

    q, kk, v = pl.pallas_call(
        _qkv_body,
        grid=(nb,),
        in_specs=[
            pl.BlockSpec((_BQ, _D), lambda i: (i, 0)),
            pl.BlockSpec((_D, _D), lambda i: (0, 0)),
            pl.BlockSpec((_D, _D), lambda i: (0, 0)),
            pl.BlockSpec((_D, _D), lambda i: (0, 0)),
            pl.BlockSpec((1, _D), lambda i: (0, 0)),
            pl.BlockSpec((1, _D), lambda i: (0, 0)),
            pl.BlockSpec((1, _D), lambda i: (0, 0)),
        ],
        out_specs=[
            pl.BlockSpec((_BQ, _D), lambda i: (i, 0)),
            pl.BlockSpec((_BQ, _D), lambda i: (i, 0)),
            pl.BlockSpec((_BQ, _D), lambda i: (i, 0)),
        ],
        out_shape=[jax.ShapeDtypeStruct((_S, _D), jnp.float32)] * 3,
    )(x2, wqt, wkt, wvt, bq2, bk2, bv2)

    q3 = q.reshape(_S, _H, _DK).transpose(1, 0, 2)
    k3 = kk.reshape(_S, _H, _DK).transpose(1, 0, 2)
    v3 = v.reshape(_S, _H, _DK).transpose(1, 0, 2)

    attn, ctx = pl.pallas_call(
        _attn_body,
        grid=(_H, nb),
        in_specs=[
            pl.BlockSpec((1, _BQ, _DK), lambda h, i: (h, i, 0)),
            pl.BlockSpec((1, _S, _DK), lambda h, i: (h, 0, 0)),
            pl.BlockSpec((1, _S, _DK), lambda h, i: (h, 0, 0)),
        ],
        out_specs=[
            pl.BlockSpec((1, _BQ, _S), lambda h, i: (h, i, 0)),
            pl.BlockSpec((1, _BQ, _DK), lambda h, i: (h, i, 0)),
        ],
        out_shape=[
            jax.ShapeDtypeStruct((_H, _S, _S), jnp.float32),
            jax.ShapeDtypeStruct((_H, _S, _DK), jnp.float32),
        ],
    )(q3, k3, v3)

    ctx2 = ctx.transpose(1, 0, 2).reshape(_S, _D)

    out = pl.pallas_call(
        _proj_body,
        grid=(nb,),
        in_specs=[
            pl.BlockSpec((_BQ, _D), lambda i: (i, 0)),
            pl.BlockSpec((_D, _D), lambda i: (0, 0)),
            pl.BlockSpec((1, _D), lambda i: (0, 0)),
        ],
        out_specs=pl.BlockSpec((_BQ, _D), lambda i: (i, 0)),
        out_shape=jax.ShapeDtypeStruct((_S, _D), jnp.float32),
    )(ctx2, wot, bo2)

    return (out.reshape(1, _S, _D), attn.reshape(1, _H, _S, _S))


# speedup vs baseline: 3.2889x; 1.0693x over previous
import math

import jax
import jax.numpy as jnp
from jax.experimental import pallas as pl

_D = 1024
_H = 16
_DK = _D // _H
_S = 2048
_K = max(1, int(_S * 0.3))
_SCALE = 1.0 / math.sqrt(_DK)
_BQ = 512
_INT_MIN = -(2 ** 31)


def _qkv_body(x_ref, wq_ref, wk_ref, wv_ref, bq_ref, bk_ref, bv_ref,
              q_ref, k_ref, v_ref):
    x = x_ref[...]
    q_ref[...] = jnp.dot(x, wq_ref[...], preferred_element_type=jnp.float32) + bq_ref[...]
    k_ref[...] = jnp.dot(x, wk_ref[...], preferred_element_type=jnp.float32) + bk_ref[...]
    v_ref[...] = jnp.dot(x, wv_ref[...], preferred_element_type=jnp.float32) + bv_ref[...]


def _attn_body(q_ref, k_ref, v_ref, attn_ref, ctx_ref):
    q = q_ref[0]
    k = k_ref[0]
    s = jax.lax.dot_general(
        q, k, (((1,), (1,)), ((), ())),
        preferred_element_type=jnp.float32) * _SCALE

    m = jnp.max(s, axis=1, keepdims=True)
    lo0 = jnp.min(s, axis=1, keepdims=True)
    scale = 524280.0 / jnp.maximum(m - lo0, jnp.float32(1e-37))
    q = ((s - lo0) * scale).astype(jnp.int32)
    qh = q >> 4
    half = _BQ // 2
    wg = ((qh[:half] << 16) | qh[half:]) | jnp.int32(-0x7FFF8000)

    tw = jnp.zeros((half, 1), jnp.int32)
    for b in range(14, -1, -1):
        trial = tw | jnp.int32((1 << (b + 16)) | (1 << b))
        d = wg - trial
        g = jax.lax.shift_right_logical(d, 15) & jnp.int32(0x00010001)
        w = _S // 2
        while w >= 128:
            g = g[:, :w] + g[:, w:]
            w //= 2
        red = jnp.sum(g, axis=1, keepdims=True)
        bit_hi = jnp.where(red >> 16 >= _K, jnp.int32(1 << (b + 16)), 0)
        bit_lo = jnp.where((red & 0xFFFF) >= _K, jnp.int32(1 << b), 0)
        tw = tw | bit_hi | bit_lo

    thrq = jnp.concatenate([tw >> 16, tw & 0xFFFF], axis=0) << 4
    for b in range(3, -1, -1):
        trial = thrq | jnp.int32(1 << b)
        cnt = jnp.sum((q >= trial).astype(jnp.float32), axis=1, keepdims=True)
        thrq = jnp.where(cnt >= jnp.float32(_K), trial, thrq)

    mask = q >= thrq
    p = jnp.where(mask, jnp.exp(s - m), 0.0)
    denom = jnp.sum(p, axis=1, keepdims=True)
    a = p / denom
    attn_ref[0] = a
    ctx_ref[0] = jnp.dot(a, v_ref[0], preferred_element_type=jnp.float32)


def _proj_body(c_ref, w_ref, b_ref, o_ref):
    o_ref[...] = jnp.dot(c_ref[...], w_ref[...],
                         preferred_element_type=jnp.float32) + b_ref[...]


def kernel(x, W_q, b_q, W_k, b_k, W_v, b_v, W_o, b_o):
    x2 = x.reshape(_S, _D)
    wqt, wkt, wvt, wot = W_q.T, W_k.T, W_v.T, W_o.T
    bq2 = b_q.reshape(1, _D)
    bk2 = b_k.reshape(1, _D)
    bv2 = b_v.reshape(1, _D)
    bo2 = b_o.reshape(1, _D)
    nb = _S // _BQ

    q, kk, v = pl.pallas_call(
        _qkv_body,
        grid=(nb,),
        in_specs=[
            pl.BlockSpec((_BQ, _D), lambda i: (i, 0)),
            pl.BlockSpec((_D, _D), lambda i: (0, 0)),
            pl.BlockSpec((_D, _D), lambda i: (0, 0)),
            pl.BlockSpec((_D, _D), lambda i: (0, 0)),
            pl.BlockSpec((1, _D), lambda i: (0, 0)),
            pl.BlockSpec((1, _D), lambda i: (0, 0)),
            pl.BlockSpec((1, _D), lambda i: (0, 0)),
        ],
        out_specs=[
            pl.BlockSpec((_BQ, _D), lambda i: (i, 0)),
            pl.BlockSpec((_BQ, _D), lambda i: (i, 0)),
            pl.BlockSpec((_BQ, _D), lambda i: (i, 0)),
        ],
        out_shape=[jax.ShapeDtypeStruct((_S, _D), jnp.float32)] * 3,
    )(x2, wqt, wkt, wvt, bq2, bk2, bv2)

    q3 = q.reshape(_S, _H, _DK).transpose(1, 0, 2)
    k3 = kk.reshape(_S, _H, _DK).transpose(1, 0, 2)
    v3 = v.reshape(_S, _H, _DK).transpose(1, 0, 2)

    attn, ctx = pl.pallas_call(
        _attn_body,
        grid=(_H, nb),
        in_specs=[
            pl.BlockSpec((1, _BQ, _DK), lambda h, i: (h, i, 0)),
            pl.BlockSpec((1, _S, _DK), lambda h, i: (h, 0, 0)),
            pl.BlockSpec((1, _S, _DK), lambda h, i: (h, 0, 0)),
        ],
        out_specs=[
            pl.BlockSpec((1, _BQ, _S), lambda h, i: (h, i, 0)),
            pl.BlockSpec((1, _BQ, _DK), lambda h, i: (h, i, 0)),
        ],
        out_shape=[
            jax.ShapeDtypeStruct((_H, _S, _S), jnp.float32),
            jax.ShapeDtypeStruct((_H, _S, _DK), jnp.float32),
        ],
    )(q3, k3, v3)

    ctx2 = ctx.transpose(1, 0, 2).reshape(_S, _D)

    out = pl.pallas_call(
        _proj_body,
        grid=(nb,),
        in_specs=[
            pl.BlockSpec((_BQ, _D), lambda i: (i, 0)),
            pl.BlockSpec((_D, _D), lambda i: (0, 0)),
            pl.BlockSpec((1, _D), lambda i: (0, 0)),
        ],
        out_specs=pl.BlockSpec((_BQ, _D), lambda i: (i, 0)),
        out_shape=jax.ShapeDtypeStruct((_S, _D), jnp.float32),
    )(ctx2, wot, bo2)

    return (out.reshape(1, _S, _D), attn.reshape(1, _H, _S, _S))
